# baseline (speedup 1.0000x reference)
import jax
import jax.numpy as jnp
from jax import lax
from jax.experimental import pallas as pl
from jax.experimental.pallas import tpu as pltpu
from jax.experimental.pallas import tpu_sc as plsc

_NC = 2
_NS = 16
_NW = _NC * _NS
_L = 16
_D = 32
_B = 16384
_BPW = _B // _NW
_CH = 128
_NCH = _BPW // _CH
_GPC = _CH // _L
_CHD = _CH * _D


def _dot_body(p1_hbm, p2_hbm, Pf_hbm, Qf_hbm, out_hbm,
              idx1_v, idx2_v, ip_v, iq_v, pg_v, qg_v, out_v,
              semp0, semp1, semq0, semq1):
    wid = lax.axis_index("s") * _NC + lax.axis_index("c")
    base = wid * _BPW
    psems = (semp0, semp1)
    qsems = (semq0, semq1)

    pltpu.sync_copy(p1_hbm.at[pl.ds(base, _BPW)], idx1_v)
    pltpu.sync_copy(p2_hbm.at[pl.ds(base, _BPW)], idx2_v)

    def build(c):
        buf = c % 2

        def row(j, carry):
            for k in range(_GPC):
                sl = pl.ds(c * _CH + k * _L, _L)
                dst = pl.ds(j * _CH + k * _L, _L)
                ip_v[buf, dst] = idx1_v[sl] * _D + j
                iq_v[buf, dst] = idx2_v[sl] * _D + j
            return carry

        lax.fori_loop(0, _D, row, 0)

    def fire(c):
        buf = c % 2

        def row(j, carry):
            sl = pl.ds(j * _CH, _CH)
            pltpu.async_copy(Pf_hbm.at[ip_v.at[buf].at[sl]],
                             pg_v.at[buf].at[sl], psems[buf])
            pltpu.async_copy(Qf_hbm.at[iq_v.at[buf].at[sl]],
                             qg_v.at[buf].at[sl], qsems[buf])
            return carry

        lax.fori_loop(0, _D, row, 0)

    def drain(c):
        buf = c % 2
        pltpu.make_async_copy(Pf_hbm.at[pl.ds(0, _CHD)], pg_v.at[buf],
                              psems[buf]).wait()
        pltpu.make_async_copy(Qf_hbm.at[pl.ds(0, _CHD)], qg_v.at[buf],
                              qsems[buf]).wait()

    build(0)
    fire(0)
    build(1)
    fire(1)

    for c in range(_NCH):
        buf = c % 2
        drain(c)

        def group(g, carry):
            e0 = g * _L
            acc = pg_v[buf, pl.ds(e0, _L)] * qg_v[buf, pl.ds(e0, _L)]
            for j in range(1, _D):
                sl = pl.ds(j * _CH + e0, _L)
                acc = acc + pg_v[buf, sl] * qg_v[buf, sl]
            out_v[pl.ds(c * _CH + e0, _L)] = acc
            return carry

        lax.fori_loop(0, _GPC, group, 0)

        if c + 2 < _NCH:
            build(c + 2)
            fire(c + 2)

    pltpu.sync_copy(out_v, out_hbm.at[pl.ds(base, _BPW)])


def _bias_body(p1_hbm, p2_hbm, b1_hbm, b2_hbm, out_hbm,
               idx1_v, idx2_v, b1_v, b2_v, out_v, sem):
    wid = lax.axis_index("s") * _NC + lax.axis_index("c")
    nrow = _BPW // 128

    pltpu.sync_copy(p1_hbm.at[pl.ds(wid * nrow, nrow)], idx1_v)
    pltpu.sync_copy(p2_hbm.at[pl.ds(wid * nrow, nrow)], idx2_v)

    copies = []
    for c in range(nrow):
        sl = pl.ds(c * 128, 128)
        copies.append(pltpu.async_copy(b1_hbm.at[idx1_v.at[c]], b1_v.at[sl], sem))
        copies.append(pltpu.async_copy(b2_hbm.at[idx2_v.at[c]], b2_v.at[sl], sem))
    for cp in copies:
        cp.wait()

    for g in range(_BPW // _L):
        sl = pl.ds(g * _L, _L)
        out_v[sl] = b1_v[sl] + b2_v[sl]

    pltpu.sync_copy(out_v, out_hbm.at[pl.ds(wid * _BPW, _BPW)])


@jax.jit
def kernel(player1, player2, P, Q, player1_bias, player2_bias):
    p1 = player1.astype(jnp.int32)
    p2 = player2.astype(jnp.int32)
    b1 = player1_bias.reshape(-1)
    b2 = player2_bias.reshape(-1)
    mesh = plsc.VectorSubcoreMesh(core_axis_name="c", subcore_axis_name="s")

    dot_f = pl.kernel(
        _dot_body,
        out_type=jax.ShapeDtypeStruct((_B,), jnp.float32),
        mesh=mesh,
        compiler_params=pltpu.CompilerParams(
            needs_layout_passes=False, use_tc_tiling_on_sc=False),
        scratch_types=[
            pltpu.VMEM((_BPW,), jnp.int32),
            pltpu.VMEM((_BPW,), jnp.int32),
            pltpu.VMEM((2, _CHD), jnp.int32),
            pltpu.VMEM((2, _CHD), jnp.int32),
            pltpu.VMEM((2, _CHD), jnp.float32),
            pltpu.VMEM((2, _CHD), jnp.float32),
            pltpu.VMEM((_BPW,), jnp.float32),
            pltpu.SemaphoreType.DMA,
            pltpu.SemaphoreType.DMA,
            pltpu.SemaphoreType.DMA,
            pltpu.SemaphoreType.DMA,
        ],
    )

    bias_f = pl.kernel(
        _bias_body,
        out_type=jax.ShapeDtypeStruct((_B,), jnp.float32),
        mesh=mesh,
        compiler_params=pltpu.CompilerParams(
            needs_layout_passes=False, use_tc_tiling_on_sc=False),
        scratch_types=[
            pltpu.VMEM((_BPW // 128, 128), jnp.int32),
            pltpu.VMEM((_BPW // 128, 128), jnp.int32),
            pltpu.VMEM((_BPW,), jnp.float32),
            pltpu.VMEM((_BPW,), jnp.float32),
            pltpu.VMEM((_BPW,), jnp.float32),
            pltpu.SemaphoreType.DMA,
        ],
    )

    dots = dot_f(p1, p2, P.reshape(-1), Q.reshape(-1))
    biases = bias_f(p1.reshape(_NW * (_BPW // 128), 128),
                    p2.reshape(_NW * (_BPW // 128), 128), b1, b2)
    return dots + biases
